# Initial kernel scaffold; baseline (speedup 1.0000x reference)
import jax
import jax.numpy as jnp
from jax.experimental import pallas as pl


def kernel(node_features, edge_index, W_emb, b_emb, Wg0, bg0, Wg1, bg1, Wg2, bg2, Wm1, bm1, Wm2, bm2, Wp1, bp1, Wp2, bp2):
    raise NotImplementedError("write your pallas kernel here")


# speedup vs baseline: 1.1921x; 1.1921x over previous
import functools

import jax
import jax.numpy as jnp
from jax import lax
from jax.experimental import pallas as pl
from jax.experimental.pallas import tpu as pltpu
from jax.experimental.pallas import tpu_sc as plsc

F32 = jnp.float32

_NC = 2
_NS = 16
_NW = _NC * _NS
_CH = 128
_BR = 1024


def _tc_first_body(x_ref, w_ref, b_ref, wt_ref, wb_ref, bm1_ref,
                   h_ref, a_ref, bt_ref):
  x = x_ref[...]
  h = jnp.maximum(jnp.dot(x, w_ref[...], preferred_element_type=F32)
                  + b_ref[...], 0.0)
  h_ref[...] = h
  a_ref[...] = jnp.dot(h, wt_ref[...], preferred_element_type=F32)
  bt_ref[...] = (jnp.dot(h, wb_ref[...], preferred_element_type=F32)
                 + bm1_ref[...])


def _tc_mid_body(x_ref, m_ref, w_ref, b_ref, wt_ref, wb_ref, bm1_ref,
                 h_ref, a_ref, bt_ref):
  x = x_ref[...] + m_ref[0] + m_ref[1]
  h = jnp.maximum(jnp.dot(x, w_ref[...], preferred_element_type=F32)
                  + b_ref[...], 0.0)
  h_ref[...] = h
  a_ref[...] = jnp.dot(h, wt_ref[...], preferred_element_type=F32)
  bt_ref[...] = (jnp.dot(h, wb_ref[...], preferred_element_type=F32)
                 + bm1_ref[...])


def _tc_final_body(x_ref, m_ref, w_ref, b_ref, wp1_ref, bp1_ref,
                   wp2_ref, bp2_ref, h_ref, pred_ref):
  x = x_ref[...] + m_ref[0] + m_ref[1]
  h = jnp.maximum(jnp.dot(x, w_ref[...], preferred_element_type=F32)
                  + b_ref[...], 0.0)
  h_ref[...] = h

  @pl.when(pl.program_id(0) == 0)
  def _():
    r = h[0:1, :]
    p = jnp.maximum(jnp.dot(r, wp1_ref[...], preferred_element_type=F32)
                    + bp1_ref[...], 0.0)
    pred_ref[...] = (jnp.dot(p, wp2_ref[...], preferred_element_type=F32)
                     + bp2_ref[...])


def _row_spec(h):
  return pl.BlockSpec((_BR, h), lambda i: (i, 0))


def _full_spec(r, c):
  return pl.BlockSpec((r, c), lambda i: (0, 0))


def _node_first(xp, W, b, wt, wb, bm1, np_rows, hdim):
  grid = (np_rows // _BR,)
  return pl.pallas_call(
      _tc_first_body,
      grid=grid,
      in_specs=[_row_spec(hdim), _full_spec(hdim, hdim), _full_spec(1, hdim),
                _full_spec(hdim, hdim), _full_spec(hdim, hdim),
                _full_spec(1, hdim)],
      out_specs=[_row_spec(hdim), _row_spec(hdim), _row_spec(hdim)],
      out_shape=[jax.ShapeDtypeStruct((np_rows, hdim), F32),
                 jax.ShapeDtypeStruct((np_rows, hdim), F32),
                 jax.ShapeDtypeStruct((np_rows, hdim), F32)],
  )(xp, W, b, wt, wb, bm1)


def _node_mid(h_prev, msg, W, b, wt, wb, bm1, np_rows, hdim):
  grid = (np_rows // _BR,)
  msg_spec = pl.BlockSpec((2, _BR, hdim), lambda i: (0, i, 0))
  return pl.pallas_call(
      _tc_mid_body,
      grid=grid,
      in_specs=[_row_spec(hdim), msg_spec, _full_spec(hdim, hdim),
                _full_spec(1, hdim), _full_spec(hdim, hdim),
                _full_spec(hdim, hdim), _full_spec(1, hdim)],
      out_specs=[_row_spec(hdim), _row_spec(hdim), _row_spec(hdim)],
      out_shape=[jax.ShapeDtypeStruct((np_rows, hdim), F32),
                 jax.ShapeDtypeStruct((np_rows, hdim), F32),
                 jax.ShapeDtypeStruct((np_rows, hdim), F32)],
  )(h_prev, msg, W, b, wt, wb, bm1)


def _node_final(h_prev, msg, W, b, wp1, bp1, wp2p, bp2p, np_rows, hdim):
  grid = (np_rows // _BR,)
  msg_spec = pl.BlockSpec((2, _BR, hdim), lambda i: (0, i, 0))
  return pl.pallas_call(
      _tc_final_body,
      grid=grid,
      in_specs=[_row_spec(hdim), msg_spec, _full_spec(hdim, hdim),
                _full_spec(1, hdim), _full_spec(hdim, hdim),
                _full_spec(1, hdim), _full_spec(hdim, hdim),
                _full_spec(1, hdim)],
      out_specs=[_row_spec(hdim), _full_spec(1, hdim)],
      out_shape=[jax.ShapeDtypeStruct((np_rows, hdim), F32),
                 jax.ShapeDtypeStruct((1, hdim), F32)],
  )(h_prev, msg, W, b, wp1, bp1, wp2p, bp2p)


def _sc_edge_body(np_rows, nchunks,
                  a_hbm, bt_hbm, h_hbm, src_hbm, dst_hbm, wm2_hbm, bm2_hbm,
                  msg_hbm, masks_hbm,
                  src_v, dst_v, a_v, b_v, masks_v, wm2_v, bm2_v,
                  msg_sh, sem1, sem2):
  cid = lax.axis_index("c")
  sid = lax.axis_index("s")
  wid = cid * _NS + sid

  pltpu.sync_copy(wm2_hbm, wm2_v)
  pltpu.sync_copy(bm2_hbm, bm2_v)

  def _zero_row(r, carry):
    for k in range(8):
      a_v[r, pl.ds(k * 16, 16)] = jnp.zeros((16,), F32)
    return carry
  lax.fori_loop(0, _CH, _zero_row, 0)

  rows_per_tile = np_rows // _NS
  for j in range(rows_per_tile // _CH):
    pltpu.sync_copy(a_v, msg_sh.at[pl.ds(sid * rows_per_tile + j * _CH, _CH)])
  plsc.subcore_barrier()

  bm2r = bm2_v[...]
  wm2r = [wm2_v[pl.ds(kk * 16, 16)] for kk in range(8)]

  def _chunk(c, carry):
    base = (wid * nchunks + c) * _CH
    pltpu.sync_copy(src_hbm.at[pl.ds(base, _CH)], src_v)
    pltpu.sync_copy(dst_hbm.at[pl.ds(base, _CH)], dst_v)
    cp1 = pltpu.async_copy(a_hbm.at[src_v], a_v, sem1)
    cp2 = pltpu.async_copy(bt_hbm.at[dst_v], b_v, sem2)
    cp1.wait()
    cp2.wait()

    def _grp(g, carry2):
      row_idx = g * 16 + lax.iota(jnp.int32, 16)
      acc = jnp.zeros((16,), F32)
      for k in range(128):
        col = jnp.full((16,), k, jnp.int32)
        a = plsc.load_gather(a_v, [row_idx, col])
        bb = plsc.load_gather(b_v, [row_idx, col])
        z = jnp.maximum(a + bb, 0.0)
        acc = acc + z * wm2r[k // 16][k % 16]
      t = acc + bm2r
      imp = 1.0 / (1.0 + jnp.exp(-t))
      mval = 1.0 / (1.0 + jnp.exp(0.8 - 2.0 * imp))
      masks_v[pl.ds(g * 16, 16)] = mval
      return carry2
    lax.fori_loop(0, _CH // 16, _grp, 0)

    cp3 = pltpu.async_copy(h_hbm.at[src_v], a_v, sem1)
    cp3.wait()

    def _scale(g, carry2):
      mm = masks_v[pl.ds(g * 16, 16)]
      for j in range(16):
        e = g * 16 + j
        m = mm[j]
        for k in range(8):
          b_v[e, pl.ds(k * 16, 16)] = a_v[e, pl.ds(k * 16, 16)] * m
      return carry2
    lax.fori_loop(0, _CH // 16, _scale, 0)

    pltpu.sync_copy(b_v, msg_sh.at[dst_v], add=True)
    pltpu.sync_copy(masks_v, masks_hbm.at[pl.ds(base, _CH)])
    return carry
  lax.fori_loop(0, nchunks, _chunk, 0)

  plsc.subcore_barrier()
  for j in range(rows_per_tile // _CH):
    rows = pl.ds(sid * rows_per_tile + j * _CH, _CH)
    pltpu.sync_copy(msg_sh.at[rows], msg_hbm.at[cid].at[rows])


def _sc_edge(a, bt, hcur, srcp, dstp, wm2f, bm2v, np_rows, ep, nchunks, hdim):
  mesh = plsc.VectorSubcoreMesh(core_axis_name="c", subcore_axis_name="s",
                                num_cores=_NC, num_subcores=_NS)
  body = functools.partial(_sc_edge_body, np_rows, nchunks)
  fn = pl.kernel(
      body,
      out_type=[jax.ShapeDtypeStruct((_NC, np_rows, hdim), F32),
                jax.ShapeDtypeStruct((ep,), F32)],
      mesh=mesh,
      compiler_params=pltpu.CompilerParams(use_tc_tiling_on_sc=False,
                                           needs_layout_passes=False),
      scratch_types=[
          pltpu.VMEM((_CH,), jnp.int32),
          pltpu.VMEM((_CH,), jnp.int32),
          pltpu.VMEM((_CH, hdim), F32),
          pltpu.VMEM((_CH, hdim), F32),
          pltpu.VMEM((_CH,), F32),
          pltpu.VMEM((hdim,), F32),
          pltpu.VMEM((16,), F32),
          pltpu.VMEM_SHARED((np_rows, hdim), F32),
          pltpu.SemaphoreType.DMA,
          pltpu.SemaphoreType.DMA,
      ],
  )
  return fn(a, bt, hcur, srcp, dstp, wm2f, bm2v)


def kernel(node_features, edge_index, W_emb, b_emb, Wg0, bg0, Wg1, bg1,
           Wg2, bg2, Wm1, bm1, Wm2, bm2, Wp1, bp1, Wp2, bp2):
  n, d = node_features.shape
  e = edge_index.shape[1]
  h = W_emb.shape[1]

  np_rows = -(-n // (_NS * _CH)) * (_NS * _CH)
  nchunks = -(-e // (_NW * _CH))
  ep = _NW * _CH * nchunks

  xp = jnp.zeros((np_rows, d), F32).at[:n].set(node_features)
  srcp = jnp.pad(edge_index[0], (0, ep - e))
  dstp = jnp.pad(edge_index[1], (0, ep - e), constant_values=n)

  wm1t = Wm1[:h]
  wm1b = Wm1[h:]
  wm2f = Wm2[:, 0]
  bm2v = jnp.full((16,), bm2[0], F32)

  b_emb2 = b_emb.reshape(1, h)
  bg0_2 = bg0.reshape(1, h)
  bg1_2 = bg1.reshape(1, h)
  bg2_2 = bg2.reshape(1, h)
  bm1_2 = bm1.reshape(1, h)
  bp1_2 = bp1.reshape(1, h)
  wp2p = jnp.zeros((h, h), F32).at[:, 0:1].set(Wp2)
  bp2p = jnp.zeros((1, h), F32).at[0, 0].set(bp2[0])

  h0, a0, b0 = _node_first(xp, W_emb, b_emb2, wm1t, wm1b, bm1_2, np_rows, h)
  msg0, _ = _sc_edge(a0, b0, h0, srcp, dstp, wm2f, bm2v, np_rows, ep,
                     nchunks, h)
  h1, a1, b1 = _node_mid(h0, msg0, Wg0, bg0_2, wm1t, wm1b, bm1_2, np_rows, h)
  msg1, _ = _sc_edge(a1, b1, h1, srcp, dstp, wm2f, bm2v, np_rows, ep,
                     nchunks, h)
  h2, a2, b2 = _node_mid(h1, msg1, Wg1, bg1_2, wm1t, wm1b, bm1_2, np_rows, h)
  msg2, masks = _sc_edge(a2, b2, h2, srcp, dstp, wm2f, bm2v, np_rows, ep,
                         nchunks, h)
  h3, predf = _node_final(h2, msg2, Wg2, bg2_2, Wp1, bp1_2, wp2p, bp2p,
                          np_rows, h)

  return (predf[0, :1], masks[:e], h3[:n])


# speedup vs baseline: 1.8821x; 1.5788x over previous
import functools

import jax
import jax.numpy as jnp
from jax import lax
from jax.experimental import pallas as pl
from jax.experimental.pallas import tpu as pltpu
from jax.experimental.pallas import tpu_sc as plsc

F32 = jnp.float32

_NC = 2
_NS = 16
_NW = _NC * _NS
_CH = 64
_BR = 1024


def _tc_first_body(x_ref, w_ref, b_ref, wt_ref, wb_ref, bm1_ref,
                   h_ref, a_ref, bt_ref):
  x = x_ref[...]
  h = jnp.maximum(jnp.dot(x, w_ref[...], preferred_element_type=F32)
                  + b_ref[...], 0.0)
  h_ref[...] = h
  a_ref[...] = jnp.dot(h, wt_ref[...], preferred_element_type=F32)
  bt_ref[...] = (jnp.dot(h, wb_ref[...], preferred_element_type=F32)
                 + bm1_ref[...])


def _tc_mid_body(x_ref, m_ref, w_ref, b_ref, wt_ref, wb_ref, bm1_ref,
                 h_ref, a_ref, bt_ref):
  x = x_ref[...] + m_ref[0] + m_ref[1]
  h = jnp.maximum(jnp.dot(x, w_ref[...], preferred_element_type=F32)
                  + b_ref[...], 0.0)
  h_ref[...] = h
  a_ref[...] = jnp.dot(h, wt_ref[...], preferred_element_type=F32)
  bt_ref[...] = (jnp.dot(h, wb_ref[...], preferred_element_type=F32)
                 + bm1_ref[...])


def _tc_final_body(x_ref, m_ref, w_ref, b_ref, wp1_ref, bp1_ref,
                   wp2_ref, bp2_ref, h_ref, pred_ref):
  x = x_ref[...] + m_ref[0] + m_ref[1]
  h = jnp.maximum(jnp.dot(x, w_ref[...], preferred_element_type=F32)
                  + b_ref[...], 0.0)
  h_ref[...] = h

  @pl.when(pl.program_id(0) == 0)
  def _():
    r = h[0:1, :]
    p = jnp.maximum(jnp.dot(r, wp1_ref[...], preferred_element_type=F32)
                    + bp1_ref[...], 0.0)
    pred_ref[...] = (jnp.dot(p, wp2_ref[...], preferred_element_type=F32)
                     + bp2_ref[...])


def _row_spec(h):
  return pl.BlockSpec((_BR, h), lambda i: (i, 0))


def _full_spec(r, c):
  return pl.BlockSpec((r, c), lambda i: (0, 0))


def _node_first(xp, W, b, wt, wb, bm1, np_rows, hdim):
  grid = (np_rows // _BR,)
  return pl.pallas_call(
      _tc_first_body,
      grid=grid,
      in_specs=[_row_spec(hdim), _full_spec(hdim, hdim), _full_spec(1, hdim),
                _full_spec(hdim, hdim), _full_spec(hdim, hdim),
                _full_spec(1, hdim)],
      out_specs=[_row_spec(hdim), _row_spec(hdim), _row_spec(hdim)],
      out_shape=[jax.ShapeDtypeStruct((np_rows, hdim), F32),
                 jax.ShapeDtypeStruct((np_rows, hdim), F32),
                 jax.ShapeDtypeStruct((np_rows, hdim), F32)],
  )(xp, W, b, wt, wb, bm1)


def _node_mid(h_prev, msg, W, b, wt, wb, bm1, np_rows, hdim):
  grid = (np_rows // _BR,)
  msg_spec = pl.BlockSpec((2, _BR, hdim), lambda i: (0, i, 0))
  return pl.pallas_call(
      _tc_mid_body,
      grid=grid,
      in_specs=[_row_spec(hdim), msg_spec, _full_spec(hdim, hdim),
                _full_spec(1, hdim), _full_spec(hdim, hdim),
                _full_spec(hdim, hdim), _full_spec(1, hdim)],
      out_specs=[_row_spec(hdim), _row_spec(hdim), _row_spec(hdim)],
      out_shape=[jax.ShapeDtypeStruct((np_rows, hdim), F32),
                 jax.ShapeDtypeStruct((np_rows, hdim), F32),
                 jax.ShapeDtypeStruct((np_rows, hdim), F32)],
  )(h_prev, msg, W, b, wt, wb, bm1)


def _node_final(h_prev, msg, W, b, wp1, bp1, wp2p, bp2p, np_rows, hdim):
  grid = (np_rows // _BR,)
  msg_spec = pl.BlockSpec((2, _BR, hdim), lambda i: (0, i, 0))
  return pl.pallas_call(
      _tc_final_body,
      grid=grid,
      in_specs=[_row_spec(hdim), msg_spec, _full_spec(hdim, hdim),
                _full_spec(1, hdim), _full_spec(hdim, hdim),
                _full_spec(1, hdim), _full_spec(hdim, hdim),
                _full_spec(1, hdim)],
      out_specs=[_row_spec(hdim), _full_spec(1, hdim)],
      out_shape=[jax.ShapeDtypeStruct((np_rows, hdim), F32),
                 jax.ShapeDtypeStruct((1, hdim), F32)],
  )(h_prev, msg, W, b, wp1, bp1, wp2p, bp2p)


_IB = 4
_MSG_ROWS = 10016


def _sc_edge_body(np_rows, nchunks,
                  a_hbm, bt_hbm, h_hbm, src_hbm, dst2_hbm, wm2_hbm, bm2_hbm,
                  msg_hbm, masks_hbm,
                  a0_v, a1_v, b0_v, b1_v, h0_v, h1_v,
                  srcb, dstb, maskb, wm2_v, bm2_v,
                  msg_sh,
                  sa0, sa1, sb0, sb1, sh0, sh1, semidx):
  cid = lax.axis_index("c")
  sid = lax.axis_index("s")
  wid = cid * _NS + sid
  nblk = nchunks // _IB

  av = [a0_v, a1_v]
  bv = [b0_v, b1_v]
  hv = [h0_v, h1_v]
  sa = [sa0, sa1]
  sb = [sb0, sb1]
  sh = [sh0, sh1]

  pltpu.sync_copy(wm2_hbm, wm2_v)
  pltpu.sync_copy(bm2_hbm, bm2_v)

  def _zero_row(r, carry):
    for k in range(8):
      h0_v[r, pl.ds(k * 16, 16)] = jnp.zeros((16,), F32)
    return carry
  lax.fori_loop(0, _CH, _zero_row, 0)

  rows_per_tile = _MSG_ROWS // _NS
  full = rows_per_tile // _CH
  rem = rows_per_tile - full * _CH
  for j in range(full):
    pltpu.sync_copy(h0_v,
                    msg_sh.at[pl.ds(sid * rows_per_tile + j * _CH, _CH)])
  if rem:
    pltpu.sync_copy(h0_v.at[pl.ds(0, rem)],
                    msg_sh.at[pl.ds(sid * rows_per_tile + full * _CH, rem)])
  plsc.subcore_barrier()

  bm2r = bm2_v[...]

  def _idx_src(bslot, boff):
    return srcb.at[bslot].at[pl.ds(boff * 64, 64)]

  def _gathers(c1, slot):
    blk1 = c1 // _IB
    boff1 = c1 % _IB
    bs1 = blk1 % 2
    pltpu.async_copy(a_hbm.at[_idx_src(bs1, boff1)], av[slot], sa[slot])
    pltpu.async_copy(bt_hbm.at[dstb.at[bs1].at[boff1]], bv[slot], sb[slot])
    pltpu.async_copy(h_hbm.at[_idx_src(bs1, boff1)], hv[slot], sh[slot])

  def _wait_gathers(c, slot):
    blk = c // _IB
    boff = c % _IB
    bs = blk % 2
    pltpu.make_async_copy(a_hbm.at[_idx_src(bs, boff)], av[slot],
                          sa[slot]).wait()
    pltpu.make_async_copy(bt_hbm.at[dstb.at[bs].at[boff]], bv[slot],
                          sb[slot]).wait()
    pltpu.make_async_copy(h_hbm.at[_idx_src(bs, boff)], hv[slot],
                          sh[slot]).wait()

  def _idx_load_refs(blk):
    nb = jnp.minimum(blk + 1, nblk - 1)
    gb = wid * nblk + nb
    ns = (blk + 1) % 2
    return (src_hbm.at[pl.ds(gb * (_IB * 64), _IB * 64)], srcb.at[ns],
            dst2_hbm.at[pl.ds(gb * _IB, _IB)], dstb.at[ns])

  gb0 = wid * nblk
  pltpu.sync_copy(src_hbm.at[pl.ds(gb0 * (_IB * 64), _IB * 64)], srcb.at[0])
  pltpu.sync_copy(dst2_hbm.at[pl.ds(gb0 * _IB, _IB)], dstb.at[0])
  _gathers(0, 0)

  def _step(c, p):
    q = 1 - p
    blk = c // _IB
    boff = c % _IB
    bs = blk % 2

    @pl.when(boff == 0)
    def _():
      s_src, d_src, s_dst, d_dst = _idx_load_refs(blk)
      pltpu.async_copy(s_src, d_src, semidx)
      pltpu.async_copy(s_dst, d_dst, semidx)

    @pl.when(boff == _IB - 1)
    def _():
      s_src, d_src, s_dst, d_dst = _idx_load_refs(blk)
      pltpu.make_async_copy(s_src, d_src, semidx).wait()
      pltpu.make_async_copy(s_dst, d_dst, semidx).wait()

    @pl.when(c + 1 < nchunks)
    def _():
      _gathers(c + 1, q)

    _wait_gathers(c, p)

    ap = av[p]
    bp = bv[p]
    hp = hv[p]

    def _grp(g, carry2):
      row_idx = g * 16 + lax.iota(jnp.int32, 16)

      def _feat(k0, acc):
        wvec = wm2_v[pl.ds(k0 * 16, 16)]
        for i in range(16):
          col = jnp.full((16,), k0 * 16 + i, jnp.int32)
          a = plsc.load_gather(ap, [row_idx, col])
          bb = plsc.load_gather(bp, [row_idx, col])
          z = jnp.maximum(a + bb, 0.0)
          acc = acc + z * wvec[i]
        return acc

      t = lax.fori_loop(0, 8, _feat, jnp.zeros((16,), F32)) + bm2r
      imp = 1.0 / (1.0 + jnp.exp(-t))
      mval = 1.0 / (1.0 + jnp.exp(0.8 - 2.0 * imp))
      maskb[boff, pl.ds(g * 16, 16)] = mval
      return carry2
    lax.fori_loop(0, _CH // 16, _grp, 0)

    def _scale(g, carry2):
      mm = maskb[boff, pl.ds(g * 16, 16)]
      for j in range(16):
        e = g * 16 + j
        m = mm[j]
        for k in range(8):
          hp[e, pl.ds(k * 16, 16)] = hp[e, pl.ds(k * 16, 16)] * m
      return carry2
    lax.fori_loop(0, _CH // 16, _scale, 0)

    pltpu.sync_copy(hp, msg_sh.at[dstb.at[bs].at[boff]], add=True)

    @pl.when(boff == _IB - 1)
    def _():
      mrow = (wid * nblk + blk) * _IB
      pltpu.sync_copy(maskb, masks_hbm.at[pl.ds(mrow, _IB)])

  def _pair(i, carry):
    _step(2 * i, 0)
    _step(2 * i + 1, 1)
    return carry
  lax.fori_loop(0, nchunks // 2, _pair, 0)

  plsc.subcore_barrier()
  for j in range(full):
    rows = pl.ds(sid * rows_per_tile + j * _CH, _CH)
    pltpu.sync_copy(msg_sh.at[rows], msg_hbm.at[cid].at[rows])
  if rem:
    rows = pl.ds(sid * rows_per_tile + full * _CH, rem)
    pltpu.sync_copy(msg_sh.at[rows], msg_hbm.at[cid].at[rows])


def _sc_edge(a, bt, hcur, srcp, dst2, wm2f, bm2v, np_rows, ep, nchunks, hdim):
  mesh = plsc.VectorSubcoreMesh(core_axis_name="c", subcore_axis_name="s",
                                num_cores=_NC, num_subcores=_NS)
  body = functools.partial(_sc_edge_body, np_rows, nchunks)
  fn = pl.kernel(
      body,
      out_type=[jax.ShapeDtypeStruct((_NC, np_rows, hdim), F32),
                jax.ShapeDtypeStruct((ep // 64, 64), F32)],
      mesh=mesh,
      compiler_params=pltpu.CompilerParams(use_tc_tiling_on_sc=False,
                                           needs_layout_passes=False),
      scratch_types=[
          pltpu.VMEM((_CH, hdim), F32),
          pltpu.VMEM((_CH, hdim), F32),
          pltpu.VMEM((_CH, hdim), F32),
          pltpu.VMEM((_CH, hdim), F32),
          pltpu.VMEM((_CH, hdim), F32),
          pltpu.VMEM((_CH, hdim), F32),
          pltpu.VMEM((2, _IB * 64), jnp.int32),
          pltpu.VMEM((2, _IB, 64), jnp.int32),
          pltpu.VMEM((_IB, 64), F32),
          pltpu.VMEM((hdim,), F32),
          pltpu.VMEM((16,), F32),
          pltpu.VMEM_SHARED((_MSG_ROWS, hdim), F32),
          pltpu.SemaphoreType.DMA,
          pltpu.SemaphoreType.DMA,
          pltpu.SemaphoreType.DMA,
          pltpu.SemaphoreType.DMA,
          pltpu.SemaphoreType.DMA,
          pltpu.SemaphoreType.DMA,
          pltpu.SemaphoreType.DMA,
      ],
  )
  return fn(a, bt, hcur, srcp, dst2, wm2f, bm2v)


def kernel(node_features, edge_index, W_emb, b_emb, Wg0, bg0, Wg1, bg1,
           Wg2, bg2, Wm1, bm1, Wm2, bm2, Wp1, bp1, Wp2, bp2):
  n, d = node_features.shape
  e = edge_index.shape[1]
  h = W_emb.shape[1]

  np_rows = -(-n // _BR) * _BR
  nchunks = -(-(-(-e // (_NW * _CH))) // (2 * _IB)) * (2 * _IB)
  ep = _NW * _CH * nchunks

  xp = jnp.zeros((np_rows, d), F32).at[:n].set(node_features)
  srcp = jnp.pad(edge_index[0], (0, ep - e))
  dst2 = jnp.pad(edge_index[1], (0, ep - e),
                 constant_values=n).reshape(ep // 64, 64)

  wm1t = Wm1[:h]
  wm1b = Wm1[h:]
  wm2f = Wm2[:, 0]
  bm2v = jnp.full((16,), bm2[0], F32)

  b_emb2 = b_emb.reshape(1, h)
  bg0_2 = bg0.reshape(1, h)
  bg1_2 = bg1.reshape(1, h)
  bg2_2 = bg2.reshape(1, h)
  bm1_2 = bm1.reshape(1, h)
  bp1_2 = bp1.reshape(1, h)
  wp2p = jnp.zeros((h, h), F32).at[:, 0:1].set(Wp2)
  bp2p = jnp.zeros((1, h), F32).at[0, 0].set(bp2[0])

  h0, a0, b0 = _node_first(xp, W_emb, b_emb2, wm1t, wm1b, bm1_2, np_rows, h)
  msg0, _ = _sc_edge(a0, b0, h0, srcp, dst2, wm2f, bm2v, np_rows, ep,
                     nchunks, h)
  h1, a1, b1 = _node_mid(h0, msg0, Wg0, bg0_2, wm1t, wm1b, bm1_2, np_rows, h)
  msg1, _ = _sc_edge(a1, b1, h1, srcp, dst2, wm2f, bm2v, np_rows, ep,
                     nchunks, h)
  h2, a2, b2 = _node_mid(h1, msg1, Wg1, bg1_2, wm1t, wm1b, bm1_2, np_rows, h)
  msg2, masks = _sc_edge(a2, b2, h2, srcp, dst2, wm2f, bm2v, np_rows, ep,
                         nchunks, h)
  h3, predf = _node_final(h2, msg2, Wg2, bg2_2, Wp1, bp1_2, wp2p, bp2p,
                          np_rows, h)

  return (predf[0, :1], masks.reshape(ep)[:e], h3[:n])
